# Initial kernel scaffold; baseline (speedup 1.0000x reference)
import jax
import jax.numpy as jnp
from jax.experimental import pallas as pl


def kernel(x, W):
    raise NotImplementedError("write your pallas kernel here")


# speedup vs baseline: 2.0725x; 2.0725x over previous
import functools
import math

import jax
import jax.numpy as jnp
from jax.experimental import pallas as pl
from jax.experimental.pallas import tpu as pltpu


def _routing_kernel(x_ref, w_ref, comb_ref, disp_ref, cnt_ref, laux_ref,
                    eidx_ref, count_s, me_s, *, bs, e, c, nblk, s_total):
    i = pl.program_id(0)

    @pl.when(i == 0)
    def _init():
        count_s[...] = jnp.zeros_like(count_s)
        me_s[...] = jnp.zeros_like(me_s)

    x = x_ref[...]
    w = w_ref[...]
    logits = jax.lax.dot_general(
        x, w, dimension_numbers=(((1,), (1,)), ((), ())),
        preferred_element_type=jnp.float32)

    m = jnp.max(logits, axis=1, keepdims=True)
    unnorm = jnp.exp(logits - m)
    gates = unnorm / jnp.sum(unnorm, axis=1, keepdims=True)

    colid = jax.lax.broadcasted_iota(jnp.int32, (bs, e), 1)
    gmax = jnp.max(gates, axis=1, keepdims=True)
    e_first = jnp.min(jnp.where(gates == gmax, colid, e), axis=1,
                      keepdims=True)
    onehot = (colid == e_first).astype(jnp.float32)

    r_iota = jax.lax.broadcasted_iota(jnp.int32, (bs, bs), 0)
    c_iota = jax.lax.broadcasted_iota(jnp.int32, (bs, bs), 1)
    tril = (c_iota <= r_iota).astype(jnp.float32)
    incl = jax.lax.dot_general(
        tril, onehot, dimension_numbers=(((1,), (0,)), ((), ())),
        preferred_element_type=jnp.float32)

    base = count_s[...]
    pos = base + incl - onehot
    keep = onehot * (pos < float(c)).astype(jnp.float32)
    count_s[...] = base + jnp.sum(onehot, axis=0, keepdims=True)
    me_s[...] = me_s[...] + jnp.sum(gates, axis=0, keepdims=True)

    eoh = gates * keep
    loc = jnp.sum(pos * keep, axis=1, keepdims=True)
    cid = jax.lax.broadcasted_iota(jnp.int32, (bs, c), 1)
    coh = (cid == loc.astype(jnp.int32)).astype(jnp.float32)

    comb = eoh[:, :, None] * coh[:, None, :]
    comb_ref[...] = comb
    disp_ref[...] = comb != 0.0

    ei = jnp.sum(colid.astype(jnp.float32) * keep, axis=1, keepdims=True)
    eidx_ref[...] = ei.astype(jnp.int32)

    @pl.when(i == nblk - 1)
    def _fin():
        cnt = count_s[...]
        cnt_ref[...] = cnt.astype(jnp.int32)
        me = me_s[...] / s_total
        ce = cnt / s_total
        laux_ref[...] = jnp.sum(me * ce, axis=1, keepdims=True) * float(e)


def kernel(x, W):
    s, d = x.shape
    e = W.shape[0]
    c = max(int(math.ceil(s / e * 1.0)), 8)
    bs = 256
    nblk = s // bs

    kern = functools.partial(_routing_kernel, bs=bs, e=e, c=c, nblk=nblk,
                             s_total=float(s))
    out_shape = [
        jax.ShapeDtypeStruct((s, e, c), jnp.float32),
        jax.ShapeDtypeStruct((s, e, c), jnp.bool_),
        jax.ShapeDtypeStruct((1, e), jnp.int32),
        jax.ShapeDtypeStruct((1, 1), jnp.float32),
        jax.ShapeDtypeStruct((s, 1), jnp.int32),
    ]
    in_specs = [
        pl.BlockSpec((bs, d), lambda i: (i, 0)),
        pl.BlockSpec((e, d), lambda i: (0, 0)),
    ]
    out_specs = [
        pl.BlockSpec((bs, e, c), lambda i: (i, 0, 0)),
        pl.BlockSpec((bs, e, c), lambda i: (i, 0, 0)),
        pl.BlockSpec((1, e), lambda i: (0, 0)),
        pl.BlockSpec((1, 1), lambda i: (0, 0)),
        pl.BlockSpec((bs, 1), lambda i: (i, 0)),
    ]
    comb, disp, cnt, laux, eidx = pl.pallas_call(
        kern,
        grid=(nblk,),
        in_specs=in_specs,
        out_specs=out_specs,
        out_shape=out_shape,
        scratch_shapes=[pltpu.VMEM((1, e), jnp.float32),
                        pltpu.VMEM((1, e), jnp.float32)],
    )(x, W)
    return (laux[0, 0], comb, disp, cnt[0], eidx)
